# baseline (speedup 1.0000x reference)
import jax
import jax.numpy as jnp
from jax import lax
from jax.experimental import pallas as pl
from jax.experimental.pallas import tpu as pltpu
from jax.experimental.pallas import tpu_sc as plsc

N = 100000
E = 6400000
TRIALS = 10

NC = 2
NS = 16
NW = NC * NS
L = 16

N_PAD = 100352
SLICE = N_PAD // NS
B = 128
ROWS = E // B
K = 8
ROWS_LO = 1560
CHUNKS = ROWS_LO // K + 1


def _payoff_body(belief_ref, probs_ref, u_ref, out_ref):
    mask = belief_ref[...] > 0.5
    p = probs_ref[...]
    cnt = jnp.zeros(belief_ref.shape, jnp.int32)
    for t in range(TRIALS):
        cnt += (u_ref[t] < p).astype(jnp.int32)
    out_ref[...] = jnp.where(mask, cnt + (TRIALS << 16), 0)


def _payoff_words(belief2d, probs2d, u3d):
    nblk = N_PAD // (8 * 128)
    return pl.pallas_call(
        _payoff_body,
        grid=(nblk,),
        in_specs=[
            pl.BlockSpec((8, 128), lambda i: (i, 0)),
            pl.BlockSpec((8, 128), lambda i: (i, 0)),
            pl.BlockSpec((TRIALS, 8, 128), lambda i: (0, i, 0)),
        ],
        out_specs=pl.BlockSpec((8, 128), lambda i: (i, 0)),
        out_shape=jax.ShapeDtypeStruct((N_PAD // 128, 128), jnp.int32),
    )(belief2d, probs2d, u3d)


def _merge_body(part_ref, p_ref, t_ref):
    p_ref[...] = part_ref[0] + part_ref[2]
    t_ref[...] = part_ref[1] + part_ref[3]


def _merge(parts):
    return pl.pallas_call(
        _merge_body,
        out_shape=[
            jax.ShapeDtypeStruct((N_PAD // 128, 128), jnp.float32),
            jax.ShapeDtypeStruct((N_PAD // 128, 128), jnp.float32),
        ],
    )(parts)


def _edge_kernel(words_hbm, edges_hbm, zeros_hbm,
                 out,
                 acc_p, acc_t,
                 table,
                 src_buf, dst_buf, wbuf, lo_buf, hi_buf,
                 se0, se1, se2, se3, sw0, sw1, ss0, ss1):
    c = lax.axis_index("c")
    s = lax.axis_index("s")
    w = s * NC + c
    se = (se0, se1, se2, se3)
    sw = (sw0, sw1)
    ss = (ss0, ss1)

    node_base = s * SLICE
    pltpu.sync_copy(words_hbm.at[pl.ds(node_base, SLICE)],
                    table.at[pl.ds(node_base, SLICE)])
    pltpu.sync_copy(zeros_hbm.at[pl.ds(node_base, SLICE)],
                    acc_p.at[pl.ds(node_base, SLICE)])
    pltpu.sync_copy(zeros_hbm.at[pl.ds(node_base, SLICE)],
                    acc_t.at[pl.ds(node_base, SLICE)])
    plsc.subcore_barrier()

    row_base = w * ROWS_LO + 8 * jnp.minimum(w, 10)
    max_chunk = (ROWS_LO // K) - 1 + jnp.where(w < 10, 1, 0)

    def _load(chunk, slot):
        r0 = row_base + jnp.minimum(chunk, max_chunk) * K
        pltpu.async_copy(edges_hbm.at[0, pl.ds(r0, K)], src_buf.at[slot],
                         se[slot])
        pltpu.async_copy(edges_hbm.at[1, pl.ds(r0, K)], dst_buf.at[slot],
                         se[slot])

    def _wait_load(slot):
        pltpu.make_async_copy(edges_hbm.at[0, pl.ds(0, K)], src_buf.at[slot],
                              se[slot]).wait()
        pltpu.make_async_copy(edges_hbm.at[1, pl.ds(0, K)], dst_buf.at[slot],
                              se[slot]).wait()

    SUB = 8
    KH = K // SUB

    def _drain_half(p):
        for j in range(KH):
            pltpu.make_async_copy(lo_buf.at[p, j],
                                  acc_p.at[dst_buf.at[0, j]], ss[p]).wait()
            pltpu.make_async_copy(hi_buf.at[p, j],
                                  acc_t.at[dst_buf.at[0, j]], ss[p]).wait()

    def _decode(p, j, live):
        for i in range(B // L):
            w16 = wbuf[p, j, pl.ds(i * L, L)]
            lo_buf[p, j, pl.ds(i * L, L)] = (
                w16 & 0xFFFF).astype(jnp.float32) * live
            hi_buf[p, j, pl.ds(i * L, L)] = (
                w16 >> 16).astype(jnp.float32) * live

    def _fire_gathers(slot, h, parity):
        for j in range(KH):
            pltpu.async_copy(table.at[src_buf.at[slot, h * KH + j]],
                             wbuf.at[parity, j], sw[parity])

    def _wait_gathers(slot, h, parity):
        for j in range(KH):
            pltpu.make_async_copy(table.at[src_buf.at[slot, h * KH + j]],
                                  wbuf.at[parity, j], sw[parity]).wait()

    N_HALF = SUB * CHUNKS

    def _half(H, v):
        u, h = v // SUB, v % SUB
        p = v & 1
        q = 1 - p
        _wait_gathers(u, h, p)

        @pl.when(H >= 2)
        def _():
            _drain_half(p)

        if h == 0:
            G = H // SUB

            @pl.when(G + 2 < CHUNKS)
            def _():
                _load(G + 2, (u + 2) % 4)

        un, hn = ((v + 1) // SUB) % 4, (v + 1) % SUB

        @pl.when(H + 1 < N_HALF)
        def _():
            if hn == 0:
                _wait_load(un)
            _fire_gathers(un, hn, q)

        live = jnp.where((H // SUB) <= max_chunk,
                         1.0, 0.0).astype(jnp.float32)
        for j in range(KH):
            _decode(p, j, live)

        for j in range(KH):
            pltpu.async_copy(lo_buf.at[p, j],
                             acc_p.at[dst_buf.at[u, h * KH + j]],
                             ss[p], add=True)
            pltpu.async_copy(hi_buf.at[p, j],
                             acc_t.at[dst_buf.at[u, h * KH + j]],
                             ss[p], add=True)

    _load(0, 0)
    _load(1, 1)
    _wait_load(0)
    _fire_gathers(0, 0, 0)

    def _g8(g8, carry):
        for v in range(4 * SUB):
            _half(g8 * 4 * SUB + v, v)
        return carry
    lax.fori_loop(0, N_HALF // (4 * SUB), _g8, None)

    for p in range(2):
        _drain_half(p)

    plsc.subcore_barrier()
    pltpu.sync_copy(acc_p.at[pl.ds(node_base, SLICE)],
                    out.at[c, 0, pl.ds(node_base, SLICE)])
    pltpu.sync_copy(acc_t.at[pl.ds(node_base, SLICE)],
                    out.at[c, 1, pl.ds(node_base, SLICE)])


_edge_call = pl.kernel(
    _edge_kernel,
    out_type=jax.ShapeDtypeStruct((NC, 2, N_PAD), jnp.float32),
    mesh=plsc.VectorSubcoreMesh(core_axis_name="c", subcore_axis_name="s"),
    scratch_types=[
        pltpu.MemorySpace.VMEM_SHARED((N_PAD,), jnp.float32),
        pltpu.MemorySpace.VMEM_SHARED((N_PAD,), jnp.float32),
        pltpu.MemorySpace.VMEM_SHARED((N_PAD,), jnp.int32),
        pltpu.VMEM((4, K, B), jnp.int32),
        pltpu.VMEM((4, K, B), jnp.int32),
        pltpu.VMEM((2, K // 8, B), jnp.int32),
        pltpu.VMEM((2, K // 8, B), jnp.float32),
        pltpu.VMEM((2, K // 8, B), jnp.float32),
    ] + [pltpu.SemaphoreType.DMA] * 8,
)


def kernel(belief, probs, bernoulli_uniforms, edge_index):
    pad_n = N_PAD - N
    belief2d = jnp.pad(belief, (0, pad_n)).reshape(N_PAD // 128, 128)
    probs2d = jnp.pad(probs, (0, pad_n)).reshape(N_PAD // 128, 128)
    u3d = jnp.pad(bernoulli_uniforms.T, ((0, 0), (0, pad_n))).reshape(
        TRIALS, N_PAD // 128, 128)

    words = _payoff_words(belief2d, probs2d, u3d).reshape(N_PAD)

    edges3 = edge_index.reshape(2, ROWS, B)
    zeros = jnp.zeros((N_PAD,), jnp.float32)
    parts = _edge_call(words, edges3, zeros)
    p_sum, t_sum = _merge(parts.reshape(NC * 2, N_PAD // 128, 128))
    return jnp.stack([p_sum.reshape(N_PAD)[:N],
                      t_sum.reshape(N_PAD)[:N]], axis=1)


# speedup vs baseline: 1.1571x; 1.1571x over previous
import jax
import jax.numpy as jnp
from jax import lax
from jax.experimental import pallas as pl
from jax.experimental.pallas import tpu as pltpu
from jax.experimental.pallas import tpu_sc as plsc

N = 100000
E = 6400000
TRIALS = 10

NC = 2
NS = 16
NW = NC * NS
L = 16

N_PAD = 100352
SLICE = N_PAD // NS
B = 128
ROWS = E // B
K = 8
ROWS_LO = 1560
CHUNKS = ROWS_LO // K + 1


def _payoff_body(belief_ref, probs_ref, u_ref, out_ref):
    mask = belief_ref[...] > 0.5
    p = probs_ref[...]
    cnt = jnp.zeros(belief_ref.shape, jnp.int32)
    for t in range(TRIALS):
        cnt += (u_ref[t] < p).astype(jnp.int32)
    out_ref[...] = jnp.where(mask, cnt + (TRIALS << 16), 0)


def _payoff_words(belief2d, probs2d, u3d):
    nblk = N_PAD // (8 * 128)
    return pl.pallas_call(
        _payoff_body,
        grid=(nblk,),
        in_specs=[
            pl.BlockSpec((8, 128), lambda i: (i, 0)),
            pl.BlockSpec((8, 128), lambda i: (i, 0)),
            pl.BlockSpec((TRIALS, 8, 128), lambda i: (0, i, 0)),
        ],
        out_specs=pl.BlockSpec((8, 128), lambda i: (i, 0)),
        out_shape=jax.ShapeDtypeStruct((N_PAD // 128, 128), jnp.int32),
    )(belief2d, probs2d, u3d)


def _merge_body(part_ref, p_ref, t_ref):
    p_ref[...] = part_ref[0] + part_ref[2]
    t_ref[...] = part_ref[1] + part_ref[3]


def _merge(parts):
    return pl.pallas_call(
        _merge_body,
        out_shape=[
            jax.ShapeDtypeStruct((N_PAD // 128, 128), jnp.float32),
            jax.ShapeDtypeStruct((N_PAD // 128, 128), jnp.float32),
        ],
    )(parts)


def _edge_kernel(words_hbm, edges_hbm, zeros_hbm,
                 out,
                 acc_p, acc_t,
                 table,
                 src_buf, dst_buf, wbuf, lo_buf, hi_buf,
                 se0, se1, se2, se3, sw0, sw1, ss0, ss1):
    c = lax.axis_index("c")
    s = lax.axis_index("s")
    w = s * NC + c
    se = (se0, se1, se2, se3)
    sw = (sw0, sw1)
    ss = (ss0, ss1)

    node_base = s * SLICE
    pltpu.sync_copy(words_hbm.at[pl.ds(node_base, SLICE)],
                    table.at[pl.ds(node_base, SLICE)])
    pltpu.sync_copy(zeros_hbm.at[pl.ds(node_base, SLICE)],
                    acc_p.at[pl.ds(node_base, SLICE)])
    pltpu.sync_copy(zeros_hbm.at[pl.ds(node_base, SLICE)],
                    acc_t.at[pl.ds(node_base, SLICE)])
    plsc.subcore_barrier()

    row_base = w * ROWS_LO + 8 * jnp.minimum(w, 10)
    max_chunk = (ROWS_LO // K) - 1 + jnp.where(w < 10, 1, 0)

    def _load(chunk, slot):
        r0 = row_base + jnp.minimum(chunk, max_chunk) * K
        pltpu.async_copy(edges_hbm.at[0, pl.ds(r0, K)], src_buf.at[slot],
                         se[slot])
        pltpu.async_copy(edges_hbm.at[1, pl.ds(r0, K)], dst_buf.at[slot],
                         se[slot])

    def _wait_load(slot):
        pltpu.make_async_copy(edges_hbm.at[0, pl.ds(0, K)], src_buf.at[slot],
                              se[slot]).wait()
        pltpu.make_async_copy(edges_hbm.at[1, pl.ds(0, K)], dst_buf.at[slot],
                              se[slot]).wait()

    SUB = 4
    KH = K // SUB

    def _drain_half(p):
        for j in range(KH):
            pltpu.make_async_copy(lo_buf.at[p, j],
                                  acc_p.at[dst_buf.at[0, j]], ss[p]).wait()
            pltpu.make_async_copy(hi_buf.at[p, j],
                                  acc_t.at[dst_buf.at[0, j]], ss[p]).wait()

    def _decode(p, j, live):
        for i in range(B // L):
            w16 = wbuf[p, j, pl.ds(i * L, L)]
            lo_buf[p, j, pl.ds(i * L, L)] = (
                w16 & 0xFFFF).astype(jnp.float32) * live
            hi_buf[p, j, pl.ds(i * L, L)] = (
                w16 >> 16).astype(jnp.float32) * live

    def _fire_gathers(slot, h, parity):
        for j in range(KH):
            pltpu.async_copy(table.at[src_buf.at[slot, h * KH + j]],
                             wbuf.at[parity, j], sw[parity])

    def _wait_gathers(slot, h, parity):
        for j in range(KH):
            pltpu.make_async_copy(table.at[src_buf.at[slot, h * KH + j]],
                                  wbuf.at[parity, j], sw[parity]).wait()

    N_HALF = SUB * CHUNKS

    def _half(H, v):
        u, h = v // SUB, v % SUB
        p = v & 1
        q = 1 - p
        _wait_gathers(u, h, p)

        @pl.when(H >= 2)
        def _():
            _drain_half(p)

        if h == 0:
            G = H // SUB

            @pl.when(G + 2 < CHUNKS)
            def _():
                _load(G + 2, (u + 2) % 4)

        un, hn = ((v + 1) // SUB) % 4, (v + 1) % SUB

        @pl.when(H + 1 < N_HALF)
        def _():
            if hn == 0:
                _wait_load(un)
            _fire_gathers(un, hn, q)

        live = jnp.where((H // SUB) <= max_chunk,
                         1.0, 0.0).astype(jnp.float32)
        for j in range(KH):
            _decode(p, j, live)

        for j in range(KH):
            pltpu.async_copy(lo_buf.at[p, j],
                             acc_p.at[dst_buf.at[u, h * KH + j]],
                             ss[p], add=True)
            pltpu.async_copy(hi_buf.at[p, j],
                             acc_t.at[dst_buf.at[u, h * KH + j]],
                             ss[p], add=True)

    _load(0, 0)
    _load(1, 1)
    _wait_load(0)
    _fire_gathers(0, 0, 0)

    def _g8(g8, carry):
        for v in range(4 * SUB):
            _half(g8 * 4 * SUB + v, v)
        return carry
    lax.fori_loop(0, N_HALF // (4 * SUB), _g8, None)

    for p in range(2):
        _drain_half(p)

    plsc.subcore_barrier()
    pltpu.sync_copy(acc_p.at[pl.ds(node_base, SLICE)],
                    out.at[c, 0, pl.ds(node_base, SLICE)])
    pltpu.sync_copy(acc_t.at[pl.ds(node_base, SLICE)],
                    out.at[c, 1, pl.ds(node_base, SLICE)])


_edge_call = pl.kernel(
    _edge_kernel,
    out_type=jax.ShapeDtypeStruct((NC, 2, N_PAD), jnp.float32),
    mesh=plsc.VectorSubcoreMesh(core_axis_name="c", subcore_axis_name="s"),
    scratch_types=[
        pltpu.MemorySpace.VMEM_SHARED((N_PAD,), jnp.float32),
        pltpu.MemorySpace.VMEM_SHARED((N_PAD,), jnp.float32),
        pltpu.MemorySpace.VMEM_SHARED((N_PAD,), jnp.int32),
        pltpu.VMEM((4, K, B), jnp.int32),
        pltpu.VMEM((4, K, B), jnp.int32),
        pltpu.VMEM((2, K // 4, B), jnp.int32),
        pltpu.VMEM((2, K // 4, B), jnp.float32),
        pltpu.VMEM((2, K // 4, B), jnp.float32),
    ] + [pltpu.SemaphoreType.DMA] * 8,
)


def kernel(belief, probs, bernoulli_uniforms, edge_index):
    pad_n = N_PAD - N
    belief2d = jnp.pad(belief, (0, pad_n)).reshape(N_PAD // 128, 128)
    probs2d = jnp.pad(probs, (0, pad_n)).reshape(N_PAD // 128, 128)
    u3d = jnp.pad(bernoulli_uniforms.T, ((0, 0), (0, pad_n))).reshape(
        TRIALS, N_PAD // 128, 128)

    words = _payoff_words(belief2d, probs2d, u3d).reshape(N_PAD)

    edges3 = edge_index.reshape(2, ROWS, B)
    zeros = jnp.zeros((N_PAD,), jnp.float32)
    parts = _edge_call(words, edges3, zeros)
    p_sum, t_sum = _merge(parts.reshape(NC * 2, N_PAD // 128, 128))
    return jnp.stack([p_sum.reshape(N_PAD)[:N],
                      t_sum.reshape(N_PAD)[:N]], axis=1)
